# baseline (speedup 1.0000x reference)
import functools

import numpy as np
import jax
import jax.numpy as jnp
from jax import lax
from jax.experimental import pallas as pl
from jax.experimental.pallas import tpu as pltpu
from jax.experimental.pallas import tpu_sc as plsc

D_EMBED = 64
SEQ = 200
BATCH = 4096
NC, NS, LANES = 2, 16, 16
NW = NC * NS
BBLK = BATCH // NW
NBUF = 4
NGRP = D_EMBED // LANES


def _encoding() -> np.ndarray:
    pos = np.arange(SEQ, dtype=np.float32)[:, None]
    two_i = 2.0 * np.floor(np.arange(D_EMBED, dtype=np.float32) / 2.0)[None, :]
    angles = (pos / np.power(np.float32(10000.0),
                             two_i / np.float32(D_EMBED))).astype(np.float32)
    even = (np.arange(D_EMBED) % 2) == 0
    return np.where(even[None, :], np.sin(angles), np.cos(angles)).astype(np.float32)


_ENC = _encoding()


def _sc_embed(xt_hbm, table_hbm, enc_hbm, out_hbm,
              idx_v, enc_v,
              in0, in1, in2, in3, ot0, ot1, ot2, ot3,
              gs0, gs1, gs2, gs3, ws0, ws1, ws2, ws3):
    wid = lax.axis_index("s") * NC + lax.axis_index("c")
    boff = pl.multiple_of(wid * BBLK, BBLK)
    ins = [in0, in1, in2, in3]
    outs = [ot0, ot1, ot2, ot3]
    gsems = [gs0, gs1, gs2, gs3]
    wsems = [ws0, ws1, ws2, ws3]

    pltpu.sync_copy(enc_hbm, enc_v)
    pltpu.sync_copy(xt_hbm.at[:, pl.ds(boff, BBLK)], idx_v)

    riv = [lax.broadcasted_iota(jnp.int32, (LANES,), 0) + LANES * j
           for j in range(BBLK // LANES)]

    def fire_gather(c, b):
        pltpu.async_copy(table_hbm.at[idx_v.at[c]], ins[b], gsems[b])

    def wait_gather(c, b):
        pltpu.make_async_copy(table_hbm.at[idx_v.at[c]], ins[b], gsems[b]).wait()

    def fire_write(c, b):
        pltpu.async_copy(outs[b], out_hbm.at[c, :, pl.ds(boff, BBLK)], wsems[b])

    def wait_write(c, b):
        pltpu.make_async_copy(outs[b], out_hbm.at[c, :, pl.ds(boff, BBLK)],
                              wsems[b]).wait()

    def compute(c, b):
        inb, outb = ins[b], outs[b]
        nsplat = jnp.full((LANES,), c, dtype=jnp.int32)

        def d_body(d, carry):
            dsplat = jnp.full((LANES,), d, dtype=jnp.int32)
            e = plsc.load_gather(enc_v, [nsplat, dsplat])
            for j in range(BBLK // LANES):
                g = plsc.load_gather(inb, [riv[j], dsplat])
                outb[d, pl.ds(LANES * j, LANES)] = g * 8.0 + e
            return carry

        lax.fori_loop(0, D_EMBED, d_body, 0)

    for b in range(NBUF):
        fire_gather(b, b)
    for b in range(NBUF):
        wait_gather(b, b)
        compute(b, b)
        fire_gather(b + NBUF, b)
        fire_write(b, b)

    def round_body(i, carry):
        c0 = NBUF * i
        for b in range(NBUF):
            c = c0 + b
            wait_write(c - NBUF, b)
            wait_gather(c, b)
            compute(c, b)
            fire_gather(c + NBUF, b)
            fire_write(c, b)
        return carry

    lax.fori_loop(1, SEQ // NBUF - 1, round_body, 0)

    for b in range(NBUF):
        c = SEQ - NBUF + b
        wait_write(c - NBUF, b)
        wait_gather(c, b)
        compute(c, b)
        fire_write(c, b)
    for b in range(NBUF):
        wait_write(SEQ - NBUF + b, b)


_embed_call = pl.kernel(
    _sc_embed,
    out_type=jax.ShapeDtypeStruct((SEQ, D_EMBED, BATCH), jnp.float32),
    mesh=plsc.VectorSubcoreMesh(core_axis_name="c", subcore_axis_name="s"),
    compiler_params=pltpu.CompilerParams(use_tc_tiling_on_sc=False,
                                         needs_layout_passes=False),
    scratch_types=(
        [pltpu.VMEM((SEQ, BBLK), jnp.int32),
         pltpu.VMEM((SEQ, D_EMBED), jnp.float32)]
        + [pltpu.VMEM((BBLK, D_EMBED), jnp.float32) for _ in range(NBUF)]
        + [pltpu.VMEM((D_EMBED, BBLK), jnp.float32) for _ in range(NBUF)]
        + [pltpu.SemaphoreType.DMA for _ in range(2 * NBUF)]
    ),
)


def kernel(x, table):
    xt = jnp.transpose(x)
    enc = jnp.asarray(_ENC)
    out3 = _embed_call(xt, table, enc)
    return jnp.transpose(out3, (2, 0, 1))


# speedup vs baseline: 1.8482x; 1.8482x over previous
import functools

import numpy as np
import jax
import jax.numpy as jnp
from jax import lax
from jax.experimental import pallas as pl
from jax.experimental.pallas import tpu as pltpu
from jax.experimental.pallas import tpu_sc as plsc

D_EMBED = 64
SEQ = 200
BATCH = 4096
NC, NS, LANES = 2, 16, 16
NW = NC * NS
BBLK = BATCH // NW
NBUF = 4
NGRP = D_EMBED // LANES


def _encoding() -> np.ndarray:
    pos = np.arange(SEQ, dtype=np.float32)[:, None]
    two_i = 2.0 * np.floor(np.arange(D_EMBED, dtype=np.float32) / 2.0)[None, :]
    angles = (pos / np.power(np.float32(10000.0),
                             two_i / np.float32(D_EMBED))).astype(np.float32)
    even = (np.arange(D_EMBED) % 2) == 0
    return np.where(even[None, :], np.sin(angles), np.cos(angles)).astype(np.float32)


_ENC = _encoding()


def _sc_embed(xt_hbm, table_hbm, enc_hbm, out_hbm,
              idx_v, enc_v,
              in0, in1, in2, in3, ot0, ot1, ot2, ot3,
              gs0, gs1, gs2, gs3, ws0, ws1, ws2, ws3):
    wid = lax.axis_index("s") * NC + lax.axis_index("c")
    boff = pl.multiple_of(wid * BBLK, BBLK)
    ins = [in0, in1, in2, in3]
    outs = [ot0, ot1, ot2, ot3]
    gsems = [gs0, gs1, gs2, gs3]
    wsems = [ws0, ws1, ws2, ws3]

    pltpu.sync_copy(enc_hbm, enc_v)
    pltpu.sync_copy(xt_hbm.at[:, pl.ds(boff, BBLK)], idx_v)

    colv = [lax.broadcasted_iota(jnp.int32, (LANES,), 0) + LANES * k
            for k in range(NGRP)]
    dtv = [c // 8 for c in colv]
    div = [c % 8 for c in colv]

    def fire_gather(c, b):
        pltpu.async_copy(table_hbm.at[idx_v.at[c]], ins[b], gsems[b])

    def wait_gather(c, b):
        pltpu.make_async_copy(table_hbm.at[idx_v.at[c]], ins[b], gsems[b]).wait()

    def fire_write(c, b):
        pltpu.async_copy(outs[b].at[:, :, pl.ds(0, BBLK)],
                         out_hbm.at[c, :, wid], wsems[b])

    def wait_write(c, b):
        pltpu.make_async_copy(outs[b].at[:, :, pl.ds(0, BBLK)],
                              out_hbm.at[c, :, wid], wsems[b]).wait()

    def compute(c, b):
        inb, outb = ins[b], outs[b]
        evec = [enc_v[c, pl.ds(LANES * k, LANES)] for k in range(NGRP)]

        def b_body(bb, carry):
            bsplat = jnp.full((LANES,), bb, dtype=jnp.int32)
            for k in range(NGRP):
                v = inb[bb, pl.ds(LANES * k, LANES)]
                plsc.store_scatter(outb, [dtv[k], div[k], bsplat],
                                   v * 8.0 + evec[k])
            return carry

        lax.fori_loop(0, BBLK, b_body, 0, unroll=2)

    for b in range(NBUF):
        fire_gather(b, b)
    for b in range(NBUF):
        wait_gather(b, b)
        compute(b, b)
        fire_gather(b + NBUF, b)
        fire_write(b, b)

    def round_body(i, carry):
        c0 = NBUF * i
        for b in range(NBUF):
            c = c0 + b
            wait_write(c - NBUF, b)
            wait_gather(c, b)
            compute(c, b)
            fire_gather(c + NBUF, b)
            fire_write(c, b)
        return carry

    lax.fori_loop(1, SEQ // NBUF - 1, round_body, 0)

    for b in range(NBUF):
        c = SEQ - NBUF + b
        wait_write(c - NBUF, b)
        wait_gather(c, b)
        compute(c, b)
        fire_write(c, b)
    for b in range(NBUF):
        wait_write(SEQ - NBUF + b, b)


_embed_call = pl.kernel(
    _sc_embed,
    out_type=jax.ShapeDtypeStruct((SEQ, D_EMBED // 8, BATCH // BBLK, 8, BBLK),
                                  jnp.float32),
    mesh=plsc.VectorSubcoreMesh(core_axis_name="c", subcore_axis_name="s"),
    compiler_params=pltpu.CompilerParams(use_tc_tiling_on_sc=False,
                                         needs_layout_passes=False),
    scratch_types=(
        [pltpu.VMEM((SEQ, BBLK), jnp.int32),
         pltpu.VMEM((SEQ, D_EMBED), jnp.float32)]
        + [pltpu.VMEM((BBLK, D_EMBED), jnp.float32) for _ in range(NBUF)]
        + [pltpu.VMEM((D_EMBED // 8, 8, BBLK + 1), jnp.float32)
           for _ in range(NBUF)]
        + [pltpu.SemaphoreType.DMA for _ in range(2 * NBUF)]
    ),
)


def kernel(x, table):
    xt = jnp.transpose(x)
    enc = jnp.asarray(_ENC)
    out5 = _embed_call(xt, table, enc)
    out = jnp.transpose(out5, (2, 4, 0, 1, 3)).reshape(BATCH, SEQ, D_EMBED)
    return out


# speedup vs baseline: 3.0233x; 1.6358x over previous
import functools

import numpy as np
import jax
import jax.numpy as jnp
from jax import lax
from jax.experimental import pallas as pl
from jax.experimental.pallas import tpu as pltpu
from jax.experimental.pallas import tpu_sc as plsc

D_EMBED = 64
SEQ = 200
BATCH = 4096
NC, NS, LANES = 2, 16, 16
NW = NC * NS
BBLK = BATCH // NW
NBUF = 4
NGRP = D_EMBED // LANES


def _encoding() -> np.ndarray:
    pos = np.arange(SEQ, dtype=np.float32)[:, None]
    two_i = 2.0 * np.floor(np.arange(D_EMBED, dtype=np.float32) / 2.0)[None, :]
    angles = (pos / np.power(np.float32(10000.0),
                             two_i / np.float32(D_EMBED))).astype(np.float32)
    even = (np.arange(D_EMBED) % 2) == 0
    return np.where(even[None, :], np.sin(angles), np.cos(angles)).astype(np.float32)


_ENC = _encoding()


def _sc_embed(xt_hbm, table_hbm, enc_hbm, out_hbm,
              idx_v, enc_v,
              in0, in1, in2, in3, ot0, ot1, ot2, ot3,
              gs0, gs1, gs2, gs3, ws0, ws1, ws2, ws3):
    wid = lax.axis_index("s") * NC + lax.axis_index("c")
    boff = pl.multiple_of(wid * BBLK, BBLK)
    ins = [in0, in1, in2, in3]
    outs = [ot0, ot1, ot2, ot3]
    gsems = [gs0, gs1, gs2, gs3]
    wsems = [ws0, ws1, ws2, ws3]

    pltpu.sync_copy(enc_hbm, enc_v)
    pltpu.sync_copy(xt_hbm.at[:, pl.ds(boff, BBLK)], idx_v)

    colv = [lax.broadcasted_iota(jnp.int32, (LANES,), 0) + LANES * k
            for k in range(NGRP)]
    dtv = [c // 8 for c in colv]
    div = [c % 8 for c in colv]

    def fire_gather(c, b):
        pltpu.async_copy(table_hbm.at[idx_v.at[c]], ins[b], gsems[b])

    def wait_gather(c, b):
        pltpu.make_async_copy(table_hbm.at[idx_v.at[c]], ins[b], gsems[b]).wait()

    def fire_write(c, b):
        pltpu.async_copy(outs[b].at[:, :, pl.ds(0, BBLK)],
                         out_hbm.at[c, :, wid], wsems[b])

    def wait_write(c, b):
        pltpu.make_async_copy(outs[b].at[:, :, pl.ds(0, BBLK)],
                              out_hbm.at[c, :, wid], wsems[b]).wait()

    def compute(c, b):
        inb, outb = ins[b], outs[b]
        evec = [enc_v[c, pl.ds(LANES * k, LANES)] for k in range(NGRP)]

        @functools.partial(plsc.parallel_loop, 0, BBLK, unroll=8)
        def b_body(bb):
            bsplat = jnp.full((LANES,), bb, dtype=jnp.int32)
            for k in range(NGRP):
                v = inb[bb, pl.ds(LANES * k, LANES)]
                plsc.store_scatter(outb, [dtv[k], div[k], bsplat],
                                   v * 8.0 + evec[k])

    for b in range(NBUF):
        fire_gather(b, b)
    for b in range(NBUF):
        wait_gather(b, b)
        compute(b, b)
        fire_gather(b + NBUF, b)
        fire_write(b, b)

    def round_body(i, carry):
        c0 = NBUF * i
        for b in range(NBUF):
            c = c0 + b
            wait_write(c - NBUF, b)
            wait_gather(c, b)
            compute(c, b)
            fire_gather(c + NBUF, b)
            fire_write(c, b)
        return carry

    lax.fori_loop(1, SEQ // NBUF - 1, round_body, 0)

    for b in range(NBUF):
        c = SEQ - NBUF + b
        wait_write(c - NBUF, b)
        wait_gather(c, b)
        compute(c, b)
        fire_write(c, b)
    for b in range(NBUF):
        wait_write(SEQ - NBUF + b, b)


_embed_call = pl.kernel(
    _sc_embed,
    out_type=jax.ShapeDtypeStruct((SEQ, D_EMBED // 8, BATCH // BBLK, 8, BBLK),
                                  jnp.float32),
    mesh=plsc.VectorSubcoreMesh(core_axis_name="c", subcore_axis_name="s"),
    compiler_params=pltpu.CompilerParams(use_tc_tiling_on_sc=False,
                                         needs_layout_passes=False),
    scratch_types=(
        [pltpu.VMEM((SEQ, BBLK), jnp.int32),
         pltpu.VMEM((SEQ, D_EMBED), jnp.float32)]
        + [pltpu.VMEM((BBLK, D_EMBED), jnp.float32) for _ in range(NBUF)]
        + [pltpu.VMEM((D_EMBED // 8, 8, BBLK + 1), jnp.float32)
           for _ in range(NBUF)]
        + [pltpu.SemaphoreType.DMA for _ in range(2 * NBUF)]
    ),
)


def kernel(x, table):
    xt = jnp.transpose(x)
    enc = jnp.asarray(_ENC)
    out5 = _embed_call(xt, table, enc)
    out = jnp.transpose(out5, (2, 4, 0, 1, 3)).reshape(BATCH, SEQ, D_EMBED)
    return out
